# baseline (speedup 1.0000x reference)
import functools

import jax
import jax.numpy as jnp
from jax import lax
from jax.experimental import pallas as pl
from jax.experimental.pallas import tpu as pltpu
from jax.experimental.pallas import tpu_sc as plsc

D_MODEL = 64
SCALE = 8.0

NC = 2
NS = 16
LANES = 16
NW = NC * NS


@functools.lru_cache(maxsize=None)
def _make_lookup(b, l):
    rows_per_w = b // NW
    mesh = plsc.VectorSubcoreMesh(
        core_axis_name="c", subcore_axis_name="s",
        num_cores=NC, num_subcores=NS)

    @functools.partial(
        pl.kernel,
        mesh=mesh,
        out_type=jax.ShapeDtypeStruct((b, l, D_MODEL), jnp.float32),
        scratch_types=[
            pltpu.VMEM((l,), jnp.int32),
            pltpu.VMEM((l, D_MODEL), jnp.float32),
            pltpu.SemaphoreType.DMA,
        ],
        compiler_params=pltpu.CompilerParams(use_tc_tiling_on_sc=False),
    )
    def lookup(table_hbm, idx_hbm, out_hbm, idx_v, rows_v, sem):
        wid = lax.axis_index("s") * NC + lax.axis_index("c")
        base = wid * rows_per_w

        def row_body(k, carry):
            brow = base + k
            pltpu.sync_copy(idx_hbm.at[pl.ds(brow * l, l)], idx_v)
            cp1 = pltpu.async_copy(
                table_hbm.at[idx_v.at[pl.ds(0, 128)]],
                rows_v.at[pl.ds(0, 128)], sem)
            cp2 = pltpu.async_copy(
                table_hbm.at[idx_v.at[pl.ds(128, l - 128)]],
                rows_v.at[pl.ds(128, l - 128)], sem)
            cp1.wait()
            cp2.wait()

            def scale_row(r, c2):
                for c in range(D_MODEL // LANES):
                    rows_v[r, pl.ds(c * LANES, LANES)] = (
                        rows_v[r, pl.ds(c * LANES, LANES)] * SCALE)
                return c2

            lax.fori_loop(0, l, scale_row, 0)
            pltpu.sync_copy(rows_v, out_hbm.at[brow])
            return carry

        lax.fori_loop(0, rows_per_w, row_body, 0)

    return lookup


def kernel(x, table):
    b, l = x.shape
    idx = x.reshape(b * l).astype(jnp.int32)
    return _make_lookup(b, l)(table, idx)
